# Initial kernel scaffold; baseline (speedup 1.0000x reference)
import jax
import jax.numpy as jnp
from jax.experimental import pallas as pl


def kernel(x, indices):
    raise NotImplementedError("write your pallas kernel here")


# speedup vs baseline: 1.0522x; 1.0522x over previous
import jax
import jax.numpy as jnp
from jax import lax
from jax.experimental import pallas as pl


SIZE = 32
CELLS = SIZE * SIZE


def _onehot_body(idx_ref, x_ref, out_ref):
    idx = idx_ref[0, 0, :]
    x = x_ref[0]
    iota = lax.broadcasted_iota(jnp.int32, (CELLS, idx.shape[0]), 0)
    onehot = (iota == idx[None, :]).astype(jnp.float32)
    out_ref[0] = jnp.dot(onehot, x, preferred_element_type=jnp.float32)


def kernel(x, indices):
    BS, E, F = x.shape
    idx = indices.astype(jnp.int32).reshape(BS, 1, E)
    out = pl.pallas_call(
        _onehot_body,
        grid=(BS,),
        in_specs=[
            pl.BlockSpec((1, 1, E), lambda b: (b, 0, 0)),
            pl.BlockSpec((1, E, F), lambda b: (b, 0, 0)),
        ],
        out_specs=pl.BlockSpec((1, CELLS, F), lambda b: (b, 0, 0)),
        out_shape=jax.ShapeDtypeStruct((BS, CELLS, F), jnp.float32),
    )(idx, x)
    return out.reshape(BS, SIZE, SIZE, F)


# speedup vs baseline: 1.1903x; 1.1312x over previous
import functools

import jax
import jax.numpy as jnp
from jax import lax
from jax.experimental import pallas as pl
from jax.experimental.pallas import tpu as pltpu
from jax.experimental.pallas import tpu_sc as plsc


SIZE = 32
CELLS = SIZE * SIZE
BS = 1024
E = 200
F = 64
IDX_SPLIT = 2
IDX_CHUNK = E // IDX_SPLIT
ZROWS = 128


def _make_sc_call():
    mesh = plsc.VectorSubcoreMesh(core_axis_name="c", subcore_axis_name="s")
    num_workers = mesh.num_cores * mesh.num_subcores
    b_per_w = BS // num_workers
    nc = mesh.num_cores

    @functools.partial(
        pl.kernel,
        out_type=jax.ShapeDtypeStruct((BS, CELLS, F), jnp.float32),
        mesh=mesh,
        compiler_params=pltpu.CompilerParams(use_tc_tiling_on_sc=False),
        scratch_types=[
            pltpu.VMEM((IDX_SPLIT, IDX_CHUNK), jnp.int32),
            pltpu.VMEM((IDX_SPLIT, IDX_CHUNK, F), jnp.float32),
            pltpu.VMEM_SHARED((mesh.num_subcores, CELLS, F), jnp.float32),
            pltpu.VMEM((ZROWS, F), jnp.float32),
        ],
    )
    def sc_scatter(x_hbm, idx_hbm, zeros_hbm, out_hbm, idx_v, x_v, acc_sh, zeros_v):
        sid = lax.axis_index("s")
        wid = sid * nc + lax.axis_index("c")
        acc_v = acc_sh.at[sid]
        pltpu.sync_copy(zeros_hbm, zeros_v)
        for k in range(CELLS // ZROWS):
            pltpu.sync_copy(zeros_v, acc_v.at[pl.ds(k * ZROWS, ZROWS)])

        def one_batch(i, carry):
            b = wid * b_per_w + i
            pltpu.sync_copy(idx_hbm.at[b], idx_v)
            pltpu.sync_copy(x_hbm.at[b], x_v)
            for j in range(IDX_SPLIT):
                pltpu.sync_copy(x_v.at[j], acc_v.at[idx_v.at[j]], add=True)
            pltpu.sync_copy(acc_v, out_hbm.at[b])
            for j in range(IDX_SPLIT):
                pltpu.sync_copy(
                    zeros_v.at[pl.ds(0, IDX_CHUNK)], acc_v.at[idx_v.at[j]]
                )
            return carry

        lax.fori_loop(0, b_per_w, one_batch, 0)

    return sc_scatter


def kernel(x, indices):
    idx32 = indices.astype(jnp.int32).reshape(BS, IDX_SPLIT, IDX_CHUNK)
    x4 = x.reshape(BS, IDX_SPLIT, IDX_CHUNK, F)
    zeros = jnp.zeros((ZROWS, F), jnp.float32)
    out = _make_sc_call()(x4, idx32, zeros)
    return out.reshape(BS, SIZE, SIZE, F)
